# Initial kernel scaffold; baseline (speedup 1.0000x reference)
import jax
import jax.numpy as jnp
from jax.experimental import pallas as pl


def kernel(inputs, W):
    raise NotImplementedError("write your pallas kernel here")


# speedup vs baseline: 1.4715x; 1.4715x over previous
import functools

import jax
import jax.numpy as jnp
from jax import lax
from jax.experimental import pallas as pl
from jax.experimental.pallas import tpu as pltpu
from jax.experimental.pallas import tpu_sc as plsc

DIM = 16
EPS = 1e-5
LANES = 16
NC = 2
NS = 16
NW = NC * NS
GB = 64


def _sc_arg_kernel(B, L):
    P = L - 1
    RPW = B // NW
    NG = RPW // GB
    GROWS = GB * L
    NIDX = GROWS // 128
    GOUT = GB * P
    IDXR_PW = (RPW * L) // 128

    mesh = plsc.VectorSubcoreMesh(
        core_axis_name="c", subcore_axis_name="s", num_cores=NC, num_subcores=NS
    )

    @functools.partial(
        pl.kernel,
        out_type=jax.ShapeDtypeStruct((B * P,), jnp.float32),
        mesh=mesh,
        scratch_types=[
            pltpu.VMEM((IDXR_PW, 128), jnp.int32),
            pltpu.VMEM((GROWS, DIM), jnp.float32),
            pltpu.VMEM((GOUT,), jnp.float32),
            pltpu.SemaphoreType.DMA,
        ],
        compiler_params=pltpu.CompilerParams(
            needs_layout_passes=False, use_tc_tiling_on_sc=False
        ),
    )
    def sc_kern(idx_hbm, w_hbm, out_hbm, idx_v, rows_v, out_v, sem):
        wid = lax.axis_index("s") * NC + lax.axis_index("c")
        iota = lax.iota(jnp.int32, LANES)
        dvecs = [jnp.full((LANES,), d, jnp.int32) for d in range(DIM)]

        pltpu.sync_copy(idx_hbm.at[wid], idx_v)

        def group_body(g, carry):
            copies = [
                pltpu.async_copy(
                    w_hbm.at[idx_v.at[g * NIDX + j]],
                    rows_v.at[pl.ds(j * 128, 128)],
                    sem,
                )
                for j in range(NIDX)
            ]
            for c in copies:
                c.wait()

            for sg in range(GB // LANES):
                row_base = iota * L + (sg * LANES * L)
                obase = iota * P + (sg * LANES * P - 1)

                s_list = []
                sp = [None, None, None, None]
                for d in range(DIM):
                    sv = plsc.load_gather(rows_v, [row_base, dvecs[d]])
                    s_list.append(sv)
                    j = d & 3
                    sp[j] = sv * sv if sp[j] is None else sp[j] + sv * sv
                sq_s = (sp[0] + sp[1]) + (sp[2] + sp[3])
                one_m_s = 1.0 - sq_s

                def k_body(k, c2):
                    row_idx = row_base + k
                    op = [None, None, None, None]
                    dp = [None, None, None, None]
                    for d in range(DIM):
                        ov = plsc.load_gather(rows_v, [row_idx, dvecs[d]])
                        df = ov - s_list[d]
                        j = d & 3
                        op[j] = ov * ov if op[j] is None else op[j] + ov * ov
                        dp[j] = df * df if dp[j] is None else dp[j] + df * df
                    sq_o = (op[0] + op[1]) + (op[2] + op[3])
                    sq_d = (dp[0] + dp[1]) + (dp[2] + dp[3])
                    denom = jnp.maximum(one_m_s * (1.0 - sq_o), EPS)
                    arg = 1.0 + 2.0 * sq_d / denom
                    plsc.store_scatter(out_v, [obase + k], arg)
                    return c2

                lax.fori_loop(1, L, k_body, 0)

            o0 = wid * (RPW * P) + g * GOUT
            pltpu.sync_copy(out_v, out_hbm.at[pl.ds(o0, GOUT)])
            return carry

        lax.fori_loop(0, NG, group_body, 0)

    return sc_kern


def _acosh_body(x_ref, o_ref):
    x = jnp.maximum(x_ref[...], 1.0 + EPS)
    o_ref[...] = jnp.log(x + jnp.sqrt((x - 1.0) * (x + 1.0)))


def kernel(inputs, W):
    B, L = inputs.shape
    P = L - 1
    idx3d = inputs.astype(jnp.int32).reshape(NW, (B * L) // (NW * 128), 128)
    arg_flat = _sc_arg_kernel(B, L)(idx3d, W)

    rows = (B * P) // 128
    blk = rows // 8
    arg2d = arg_flat.reshape(rows, 128)
    dist = pl.pallas_call(
        _acosh_body,
        out_shape=jax.ShapeDtypeStruct((rows, 128), jnp.float32),
        grid=(8,),
        in_specs=[pl.BlockSpec((blk, 128), lambda i: (i, 0))],
        out_specs=pl.BlockSpec((blk, 128), lambda i: (i, 0)),
    )(arg2d)
    return dist.reshape(B, P)


# speedup vs baseline: 1.5013x; 1.0202x over previous
import functools

import jax
import jax.numpy as jnp
from jax import lax
from jax.experimental import pallas as pl
from jax.experimental.pallas import tpu as pltpu
from jax.experimental.pallas import tpu_sc as plsc

DIM = 16
EPS = 1e-5
LANES = 16
NC = 2
NS = 16
NW = NC * NS
GB = 64


def _sc_arg_kernel(B, L):
    P = L - 1
    RPW = B // NW
    NG = RPW // GB
    GROWS = GB * L
    NIDX = GROWS // 128

    mesh = plsc.VectorSubcoreMesh(
        core_axis_name="c", subcore_axis_name="s", num_cores=NC, num_subcores=NS
    )

    @functools.partial(
        pl.kernel,
        out_type=jax.ShapeDtypeStruct((B, P), jnp.float32),
        mesh=mesh,
        scratch_types=[
            [pltpu.VMEM((GB, L), jnp.int32) for _ in range(2)],
            [pltpu.VMEM((NIDX, 128), jnp.int32) for _ in range(2)],
            [pltpu.VMEM((GROWS, DIM), jnp.float32) for _ in range(2)],
            [pltpu.VMEM((GB, P), jnp.float32) for _ in range(2)],
            [pltpu.SemaphoreType.DMA for _ in range(2)],
        ],
        compiler_params=pltpu.CompilerParams(
            needs_layout_passes=False, use_tc_tiling_on_sc=False
        ),
    )
    def sc_kern(idx_hbm, w_hbm, out_hbm, stag, idxp, rows, outb, sems):
        wid = lax.axis_index("s") * NC + lax.axis_index("c")
        iota = lax.iota(jnp.int32, LANES)
        dvecs = [jnp.full((LANES,), d, jnp.int32) for d in range(DIM)]

        def stage_repack(g, slot):
            b0 = wid * RPW + g * GB
            pltpu.sync_copy(idx_hbm.at[pl.ds(b0, GB)], stag[slot])

            def jbody(j, c):
                for u in range(128 // LANES):
                    q = j * 128 + (u * LANES) + iota
                    r = q // L
                    cc = q - r * L
                    v = plsc.load_gather(stag[slot], [r, cc])
                    idxp[slot][j, pl.ds(u * LANES, LANES)] = v
                return c

            lax.fori_loop(0, NIDX, jbody, 0)

        def fire(slot):
            for j in range(NIDX):
                pltpu.async_copy(
                    w_hbm.at[idxp[slot].at[j]],
                    rows[slot].at[pl.ds(j * 128, 128)],
                    sems[slot],
                )

        def drain(slot):
            pltpu.make_async_copy(
                w_hbm.at[pl.ds(0, GROWS)], rows[slot], sems[slot]
            ).wait()

        def compute(g, slot):
            for sg in range(GB // LANES):
                row_base = iota * L + (sg * LANES * L)
                orow = iota + sg * LANES

                s_list = []
                sp = [None, None, None, None]
                for d in range(DIM):
                    sv = plsc.load_gather(rows[slot], [row_base, dvecs[d]])
                    s_list.append(sv)
                    j = d & 3
                    sp[j] = sv * sv if sp[j] is None else sp[j] + sv * sv
                sq_s = (sp[0] + sp[1]) + (sp[2] + sp[3])
                one_m_s = 1.0 - sq_s

                def k_body(k, c2):
                    row_idx = row_base + k
                    op = [None, None, None, None]
                    dp = [None, None, None, None]
                    for d in range(DIM):
                        ov = plsc.load_gather(rows[slot], [row_idx, dvecs[d]])
                        df = ov - s_list[d]
                        j = d & 3
                        op[j] = ov * ov if op[j] is None else op[j] + ov * ov
                        dp[j] = df * df if dp[j] is None else dp[j] + df * df
                    sq_o = (op[0] + op[1]) + (op[2] + op[3])
                    sq_d = (dp[0] + dp[1]) + (dp[2] + dp[3])
                    denom = jnp.maximum(one_m_s * (1.0 - sq_o), EPS)
                    arg = 1.0 + 2.0 * sq_d / denom
                    kcol = jnp.full((LANES,), k - 1, jnp.int32)
                    plsc.store_scatter(outb[slot], [orow, kcol], arg)
                    return c2

                lax.fori_loop(1, L, k_body, 0)

            b0 = wid * RPW + g * GB
            pltpu.sync_copy(outb[slot], out_hbm.at[pl.ds(b0, GB)])

        stage_repack(0, 0)
        fire(0)

        def pair_body(gp, c):
            g = 2 * gp
            stage_repack(g + 1, 1)
            fire(1)
            drain(0)
            compute(g, 0)
            gnext = jnp.minimum(g + 2, NG - 1)
            stage_repack(gnext, 0)
            fire(0)
            drain(1)
            compute(g + 1, 1)
            return c

        lax.fori_loop(0, NG // 2, pair_body, 0)
        drain(0)

    return sc_kern


def _acosh_body(x_ref, o_ref):
    x = jnp.maximum(x_ref[...], 1.0 + EPS)
    o_ref[...] = jnp.log(x + jnp.sqrt((x - 1.0) * (x + 1.0)))


def kernel(inputs, W):
    B, L = inputs.shape
    P = L - 1
    arg = _sc_arg_kernel(B, L)(inputs.astype(jnp.int32), W)

    blk = B // 8
    dist = pl.pallas_call(
        _acosh_body,
        out_shape=jax.ShapeDtypeStruct((B, P), jnp.float32),
        grid=(8,),
        in_specs=[pl.BlockSpec((blk, P), lambda i: (i, 0))],
        out_specs=pl.BlockSpec((blk, P), lambda i: (i, 0)),
    )(arg)
    return dist
